# Initial kernel scaffold; baseline (speedup 1.0000x reference)
import jax
import jax.numpy as jnp
from jax.experimental import pallas as pl


def kernel(x, W_linear, bias, W_ffm):
    raise NotImplementedError("write your pallas kernel here")


# speedup vs baseline: 22.6627x; 22.6627x over previous
import functools

import jax
import jax.numpy as jnp
import numpy as np
from jax import lax
from jax.experimental import pallas as pl
from jax.experimental.pallas import tpu as pltpu
from jax.experimental.pallas import tpu_sc as plsc

_F = 26
_VD = 1000
_E = 32
_B = 4096
_NC = 2
_NS = 16
_NW = _NC * _NS
_NB = _B // _NW
_G = 16
_NG = _NB // _G
_NPAIR = 325
_HPAD = 336
_NCHUNK = 6
_CH = 112

_PI, _PJ = np.triu_indices(_F, 1)


def _ffm_body(tbl, lin_tbl, idxf, idxl, out, idxf_v, idxl_v, rows_v, lin_v,
              accv, out_v, sem):
    wid = lax.axis_index("s") * _NC + lax.axis_index("c")
    base = wid * _NB

    def group_body(g, _):
        gbase = base + g * _G
        pltpu.sync_copy(idxf.at[pl.ds(gbase, _G)], idxf_v)
        pltpu.sync_copy(idxl.at[pl.ds(gbase, _G)], idxl_v)

        def sample_body(s, _):
            copies = []
            for c in range(_NCHUNK):
                copies.append(pltpu.async_copy(
                    tbl.at[idxf_v.at[s, c]],
                    rows_v.at[pl.ds(c * _CH, _CH)], sem))
            copies.append(pltpu.async_copy(lin_tbl.at[idxl_v.at[s]], lin_v,
                                           sem))
            for cp in copies:
                cp.wait()

            zero = jnp.zeros((16,), jnp.float32)

            def pair_body(r, acc):
                a0, a1 = acc
                for u in range(5):
                    row = r * 5 + u
                    alo = rows_v[row, pl.ds(0, 16)]
                    ahi = rows_v[row, pl.ds(16, 16)]
                    blo = rows_v[_HPAD + row, pl.ds(0, 16)]
                    bhi = rows_v[_HPAD + row, pl.ds(16, 16)]
                    a0 = a0 + alo * blo
                    a1 = a1 + ahi * bhi
                return (a0, a1)

            acc0, acc1 = lax.fori_loop(0, _NPAIR // 5, pair_body, (zero, zero))

            def lin_body(r, acc):
                return acc + lin_v[r, pl.ds(0, 16)]

            accl = lax.fori_loop(0, _F, lin_body, zero)

            accv[pl.ds(s * 16, 16)] = acc0 + acc1 + accl
            return 0

        lax.fori_loop(0, _G, sample_body, 0)

        lane = lax.iota(jnp.int32, 16)
        total_vec = jnp.zeros((16,), jnp.float32)
        for c in range(16):
            total_vec = total_vec + plsc.load_gather(accv, [lane * 16 + c])
        out_v[pl.ds(g * _G, _G)] = total_vec
        return 0

    lax.fori_loop(0, _NG, group_body, 0)
    pltpu.sync_copy(out_v, out.at[pl.ds(base, _NB)])


@jax.jit
def _ffm_sc(tbl, lin_tbl, idx_ffm, idx_lin):
    mesh = plsc.VectorSubcoreMesh(core_axis_name="c", subcore_axis_name="s")
    return pl.kernel(
        _ffm_body,
        out_type=jax.ShapeDtypeStruct((_B,), jnp.float32),
        mesh=mesh,
        compiler_params=pltpu.CompilerParams(needs_layout_passes=False,
                                             use_tc_tiling_on_sc=False),
        scratch_types=[
            pltpu.VMEM((_G, _NCHUNK, _CH), jnp.int32),
            pltpu.VMEM((_G, 32), jnp.int32),
            pltpu.VMEM((2 * _HPAD, _E), jnp.float32),
            pltpu.VMEM((32, 32), jnp.float32),
            pltpu.VMEM((_G * 16,), jnp.float32),
            pltpu.VMEM((_NB,), jnp.float32),
            pltpu.SemaphoreType.DMA,
        ],
    )(tbl, lin_tbl, idx_ffm, idx_lin)


def kernel(x, W_linear, bias, W_ffm):
    pi = jnp.asarray(_PI, jnp.int32)
    pj = jnp.asarray(_PJ, jnp.int32)
    idx_a = x[:, _PI] + (pj * (_F * _VD) + pi * _VD)[None, :]
    idx_b = x[:, _PJ] + (pi * (_F * _VD) + pj * _VD)[None, :]
    pad = ((0, 0), (0, _HPAD - _NPAIR))
    idx_ffm = jnp.concatenate(
        [jnp.pad(idx_a, pad), jnp.pad(idx_b, pad)], axis=1
    ).reshape(_B, _NCHUNK, _CH)
    xi = x + (jnp.arange(_F, dtype=x.dtype) * _VD)[None, :]
    idx_lin = jnp.pad(xi, ((0, 0), (0, 32 - _F)))
    lin32 = jnp.pad(W_linear, ((0, 0), (0, 32 - 1)))
    tbl = W_ffm.reshape(_F * _F * _VD, _E)
    out = _ffm_sc(tbl, lin32, idx_ffm, idx_lin)
    return out + bias[0]


# speedup vs baseline: 22.6895x; 1.0012x over previous
import functools

import jax
import jax.numpy as jnp
import numpy as np
from jax import lax
from jax.experimental import pallas as pl
from jax.experimental.pallas import tpu as pltpu
from jax.experimental.pallas import tpu_sc as plsc

_F = 26
_VD = 1000
_E = 32
_B = 4096
_NC = 2
_NS = 16
_NW = _NC * _NS
_NB = _B // _NW
_G = 16
_NG = _NB // _G
_NPAIR = 325
_HPAD = 336
_NCHUNK = 6
_CH = 112

_PI, _PJ = np.triu_indices(_F, 1)


def _ffm_body(tbl, lin_tbl, idxf, idxl, out, idxf_v, idxl_v, rows_v, lin_v,
              accv, out_v, sem0, sem1):
    wid = lax.axis_index("s") * _NC + lax.axis_index("c")
    base = wid * _NB
    sems = (sem0, sem1)

    def stage(g):
        gp = lax.rem(g, 2)
        pltpu.sync_copy(idxf.at[pl.ds(base + g * _G, _G)], idxf_v.at[gp])
        pltpu.sync_copy(idxl.at[pl.ds(base + g * _G, _G)], idxl_v.at[gp])

    def copies(t, buf, sem):
        gp = lax.rem(lax.div(t, _G), 2)
        ls = lax.rem(t, _G)
        cps = []
        for c in range(_NCHUNK):
            cps.append(pltpu.make_async_copy(
                tbl.at[idxf_v.at[gp, ls, c]],
                rows_v.at[buf, pl.ds(c * _CH, _CH)], sem))
        cps.append(pltpu.make_async_copy(lin_tbl.at[idxl_v.at[gp, ls]],
                                         lin_v.at[buf], sem))
        return cps

    def issue(t, buf, sem):
        for cp in copies(t, buf, sem):
            cp.start()

    def drain(t, buf, sem):
        for cp in copies(t, buf, sem):
            cp.wait()

    def compute(t, buf):
        zero = jnp.zeros((16,), jnp.float32)

        def pair_body(r, acc):
            a0, a1 = acc
            for u in range(5):
                row = r * 5 + u
                alo = rows_v[buf, row, pl.ds(0, 16)]
                ahi = rows_v[buf, row, pl.ds(16, 16)]
                blo = rows_v[buf, _HPAD + row, pl.ds(0, 16)]
                bhi = rows_v[buf, _HPAD + row, pl.ds(16, 16)]
                a0 = a0 + alo * blo
                a1 = a1 + ahi * bhi
            return (a0, a1)

        acc0, acc1 = lax.fori_loop(0, _NPAIR // 5, pair_body, (zero, zero))

        def lin_body(r, acc):
            return acc + lin_v[buf, r, pl.ds(0, 16)]

        accl = lax.fori_loop(0, _F, lin_body, zero)

        ls = lax.rem(t, _G)
        accv[pl.ds(ls * 16, 16)] = acc0 + acc1 + accl

        @pl.when(ls == _G - 1)
        def _():
            lane = lax.iota(jnp.int32, 16)
            total_vec = jnp.zeros((16,), jnp.float32)
            for c in range(16):
                total_vec = total_vec + plsc.load_gather(
                    accv, [lane * 16 + c])
            out_v[pl.ds((t - (_G - 1)), _G)] = total_vec

    stage(0)
    issue(0, 0, sems[0])

    def loop2(i, _):
        for b in (0, 1):
            t = i * 2 + b
            nt = t + 1

            @pl.when(nt < _NB)
            def _():
                @pl.when(lax.rem(nt, _G) == 0)
                def _():
                    stage(lax.div(nt, _G))

                issue(nt, 1 - b, sems[1 - b])

            drain(t, b, sems[b])
            compute(t, b)
        return 0

    lax.fori_loop(0, _NB // 2, loop2, 0)
    pltpu.sync_copy(out_v, out.at[pl.ds(base, _NB)])


@jax.jit
def _ffm_sc(tbl, lin_tbl, idx_ffm, idx_lin):
    mesh = plsc.VectorSubcoreMesh(core_axis_name="c", subcore_axis_name="s")
    return pl.kernel(
        _ffm_body,
        out_type=jax.ShapeDtypeStruct((_B,), jnp.float32),
        mesh=mesh,
        compiler_params=pltpu.CompilerParams(needs_layout_passes=False,
                                             use_tc_tiling_on_sc=False),
        scratch_types=[
            pltpu.VMEM((2, _G, _NCHUNK, _CH), jnp.int32),
            pltpu.VMEM((2, _G, 32), jnp.int32),
            pltpu.VMEM((2, 2 * _HPAD, _E), jnp.float32),
            pltpu.VMEM((2, 32, 32), jnp.float32),
            pltpu.VMEM((_G * 16,), jnp.float32),
            pltpu.VMEM((_NB,), jnp.float32),
            pltpu.SemaphoreType.DMA,
            pltpu.SemaphoreType.DMA,
        ],
    )(tbl, lin_tbl, idx_ffm, idx_lin)


def kernel(x, W_linear, bias, W_ffm):
    pi = jnp.asarray(_PI, jnp.int32)
    pj = jnp.asarray(_PJ, jnp.int32)
    idx_a = x[:, _PI] + (pj * (_F * _VD) + pi * _VD)[None, :]
    idx_b = x[:, _PJ] + (pi * (_F * _VD) + pj * _VD)[None, :]
    pad = ((0, 0), (0, _HPAD - _NPAIR))
    idx_ffm = jnp.concatenate(
        [jnp.pad(idx_a, pad), jnp.pad(idx_b, pad)], axis=1
    ).reshape(_B, _NCHUNK, _CH)
    xi = x + (jnp.arange(_F, dtype=x.dtype) * _VD)[None, :]
    idx_lin = jnp.pad(xi, ((0, 0), (0, 32 - _F)))
    lin32 = jnp.pad(W_linear, ((0, 0), (0, 32 - 1)))
    tbl = W_ffm.reshape(_F * _F * _VD, _E)
    out = _ffm_sc(tbl, lin32, idx_ffm, idx_lin)
    return out + bias[0]
